# baseline (device time: 176865 ns/iter reference)
import jax
import jax.numpy as jnp
from jax import lax
from jax.experimental import pallas as pl
from jax.experimental.pallas import tpu as pltpu

N_DEV = 8
NPART = 4
_GELU_C = 0.7978845608028654


def kernel(x, w_mat):
    m_per, k = x.shape
    _, n_per = w_mat.shape
    half = m_per // 2
    part = half // NPART

    def body(x_ref, w_ref, out_ref, gather_ref,
             cw_send, cw_recv, ccw_send, ccw_recv):
        my = lax.axis_index("i")
        left = lax.rem(my - 1 + N_DEV, N_DEV)
        right = lax.rem(my + 1, N_DEV)

        barrier_sem = pltpu.get_barrier_semaphore()
        for nbr in (left, right):
            pl.semaphore_signal(
                barrier_sem, inc=1,
                device_id=(nbr,), device_id_type=pl.DeviceIdType.MESH,
            )
        pl.semaphore_wait(barrier_sem, 2)

        def o_of(d, h):
            return lax.rem(my - h + N_DEV, N_DEV) if d == 0 else \
                lax.rem(my + h, N_DEV)

        def make_rdma(d, h, j):
            o = o_of(d, h)
            send = (cw_send, ccw_send)[d]
            recv = (cw_recv, ccw_recv)[d]
            return pltpu.make_async_remote_copy(
                src_ref=gather_ref.at[d, o, j],
                dst_ref=gather_ref.at[d, o, j],
                send_sem=send.at[h, j],
                recv_sem=recv.at[h, j],
                device_id=(right if d == 0 else left,),
                device_id_type=pl.DeviceIdType.MESH,
            )

        rdmas = {}
        for j in range(NPART):
            for d in range(2):
                r0 = d * half + j * part
                gather_ref[d, my, j] = x_ref[r0:r0 + part, :].astype(jnp.bfloat16)
                rdmas[(d, 0, j)] = make_rdma(d, 0, j)
                rdmas[(d, 0, j)].start()

        w_bf = w_ref[...].astype(jnp.bfloat16)

        def compute_half(d, origin):
            chunk = gather_ref[d, origin].reshape(half, k)
            y = jnp.dot(chunk, w_bf, preferred_element_type=jnp.float32)
            y = 0.5 * y * (1.0 + jnp.tanh(_GELU_C * (y + 0.044715 * y * y * y)))
            out_ref[pl.ds(origin * m_per + d * half, half), :] = y

        def compute_part(d, origin, j):
            y = jnp.dot(gather_ref[d, origin, j], w_bf,
                        preferred_element_type=jnp.float32)
            y = 0.5 * y * (1.0 + jnp.tanh(_GELU_C * (y + 0.044715 * y * y * y)))
            out_ref[pl.ds(origin * m_per + d * half + j * part, part), :] = y

        for h in range(N_DEV - 1):
            compute_half(0, o_of(0, h))
            compute_half(1, o_of(1, h))
            for j in range(NPART):
                for d in range(2):
                    rdmas[(d, h, j)].wait_recv()
                    if h + 1 < N_DEV - 1:
                        r = make_rdma(d, h + 1, j)
                        rdmas[(d, h + 1, j)] = r
                        r.start()
                    else:
                        compute_part(d, o_of(d, N_DEV - 1), j)
            if h >= 1:
                for j in range(NPART):
                    for d in range(2):
                        rdmas.pop((d, h - 1, j)).wait_send()
        for r in rdmas.values():
            r.wait_send()

    out_shape = jax.ShapeDtypeStruct((N_DEV * m_per, n_per), jnp.float32)
    return pl.pallas_call(
        body,
        out_shape=out_shape,
        in_specs=[
            pl.BlockSpec(memory_space=pltpu.VMEM),
            pl.BlockSpec(memory_space=pltpu.VMEM),
        ],
        out_specs=pl.BlockSpec(memory_space=pltpu.VMEM),
        scratch_shapes=[
            pltpu.VMEM((2, N_DEV, NPART, part, k), jnp.bfloat16),
            pltpu.SemaphoreType.DMA((N_DEV - 1, NPART)),
            pltpu.SemaphoreType.DMA((N_DEV - 1, NPART)),
            pltpu.SemaphoreType.DMA((N_DEV - 1, NPART)),
            pltpu.SemaphoreType.DMA((N_DEV - 1, NPART)),
        ],
        compiler_params=pltpu.CompilerParams(
            collective_id=0,
            vmem_limit_bytes=100 * 1024 * 1024,
        ),
    )(x, w_mat)


# device time: 132982 ns/iter; 1.3300x vs baseline; 1.3300x over previous
import jax
import jax.numpy as jnp
from jax import lax
from jax.experimental import pallas as pl
from jax.experimental.pallas import tpu as pltpu

N_DEV = 8
_GELU_C = 0.7978845608028654

_PIECES = ((0, 176), (176, 176), (352, 160))


def _lab(p):
    low = p & 3
    return (p & 4) | (low ^ (low >> 1))


def kernel(x, w_mat):
    m_per, k = x.shape
    _, n_per = w_mat.shape

    def body(x_ref, w_ref, out_ref, chunk_buf, send_sems, recv_sems):
        my = lax.axis_index("i")
        my_lab = _lab(my)

        barrier_sem = pltpu.get_barrier_semaphore()
        for dim in range(3):
            pl.semaphore_signal(
                barrier_sem, inc=1,
                device_id=(_lab(my_lab ^ (1 << dim)),),
                device_id_type=pl.DeviceIdType.MESH,
            )
        pl.semaphore_wait(barrier_sem, 3)

        sends = []

        def send_piece(j, r, dim):
            slot = _lab(my_lab ^ r)
            nb = _lab(my_lab ^ (1 << dim))
            r0, nr = _PIECES[j]
            rd = pltpu.make_async_remote_copy(
                src_ref=chunk_buf.at[slot, pl.ds(r0, nr)],
                dst_ref=chunk_buf.at[slot, pl.ds(r0, nr)],
                send_sem=send_sems.at[len(sends)],
                recv_sem=recv_sems.at[j, r ^ (1 << dim)],
                device_id=(nb,),
                device_id_type=pl.DeviceIdType.MESH,
            )
            rd.start()
            sends.append(rd)

        def wait_piece(j, r):
            slot = _lab(my_lab ^ r)
            r0, nr = _PIECES[j]
            rd = pltpu.make_async_remote_copy(
                src_ref=chunk_buf.at[slot, pl.ds(r0, nr)],
                dst_ref=chunk_buf.at[slot, pl.ds(r0, nr)],
                send_sem=send_sems.at[0],
                recv_sem=recv_sems.at[j, r],
                device_id=(my,),
                device_id_type=pl.DeviceIdType.MESH,
            )
            rd.wait_recv()

        for j in range(3):
            r0, nr = _PIECES[j]
            chunk_buf[my, pl.ds(r0, nr)] = x_ref[r0:r0 + nr, :].astype(jnp.bfloat16)
            send_piece(j, 0, j)

        w_bf = w_ref[...].astype(jnp.bfloat16)

        def compute_chunk(r):
            slot = _lab(my_lab ^ r)
            y = jnp.dot(chunk_buf[slot], w_bf, preferred_element_type=jnp.float32)
            y = 0.5 * y * (1.0 + jnp.tanh(_GELU_C * (y + 0.044715 * y * y * y)))
            out_ref[pl.ds(slot * m_per, m_per), :] = y

        compute_chunk(0)

        for j in range(3):
            j1, j2 = (j + 1) % 3, (j + 2) % 3
            bj, b1, b2 = 1 << j, 1 << j1, 1 << j2
            wait_piece(j, bj)
            send_piece(j, bj, j1)
            send_piece(j, bj, j2)
        for j in range(3):
            j1, j2 = (j + 1) % 3, (j + 2) % 3
            bj, b1, b2 = 1 << j, 1 << j1, 1 << j2
            wait_piece(j, bj | b1)
            send_piece(j, bj | b1, j2)
            send_piece(j, bj | b1, j)
            wait_piece(j, bj | b2)
            send_piece(j, bj | b2, j)
        for j in range(3):
            wait_piece(j, 7)
            send_piece(j, 7, j)
        for j in range(3):
            j1, j2 = (j + 1) % 3, (j + 2) % 3
            wait_piece(j, 1 << j1)
            wait_piece(j, 1 << j2)
        for r in (1, 2, 4, 7):
            compute_chunk(r)
        for j, r_done in ((0, 6), (1, 5), (2, 3)):
            j1, j2 = (j + 1) % 3, (j + 2) % 3
            wait_piece(j, (1 << j1) | (1 << j2))
            compute_chunk(r_done)
        for rd in sends:
            rd.wait_send()

    out_shape = jax.ShapeDtypeStruct((N_DEV * m_per, n_per), jnp.float32)
    return pl.pallas_call(
        body,
        out_shape=out_shape,
        in_specs=[
            pl.BlockSpec(memory_space=pltpu.VMEM),
            pl.BlockSpec(memory_space=pltpu.VMEM),
        ],
        out_specs=pl.BlockSpec(memory_space=pltpu.VMEM),
        scratch_shapes=[
            pltpu.VMEM((N_DEV, m_per, k), jnp.bfloat16),
            pltpu.SemaphoreType.DMA((21,)),
            pltpu.SemaphoreType.DMA((3, 8)),
        ],
        compiler_params=pltpu.CompilerParams(
            collective_id=0,
            vmem_limit_bytes=100 * 1024 * 1024,
        ),
    )(x, w_mat)


# device time: 128761 ns/iter; 1.3736x vs baseline; 1.0328x over previous
import jax
import jax.numpy as jnp
from jax import lax
from jax.experimental import pallas as pl
from jax.experimental.pallas import tpu as pltpu

N_DEV = 8
_GELU_C = 0.7978845608028654

_PIECES = ((0, 176), (176, 176), (352, 160))


def _lab(p):
    low = p & 3
    return (p & 4) | (low ^ (low >> 1))


def kernel(x, w_mat):
    m_per, k = x.shape
    _, n_per = w_mat.shape

    def body(x_ref, w_ref, out_ref, chunk_buf, send_sems, recv_sems):
        my = lax.axis_index("i")
        my_lab = _lab(my)

        barrier_sem = pltpu.get_barrier_semaphore()
        for dim in range(3):
            pl.semaphore_signal(
                barrier_sem, inc=1,
                device_id=(_lab(my_lab ^ (1 << dim)),),
                device_id_type=pl.DeviceIdType.MESH,
            )
        pl.semaphore_wait(barrier_sem, 3)

        sends = []

        def send_piece(j, r, dim):
            slot = _lab(my_lab ^ r)
            nb = _lab(my_lab ^ (1 << dim))
            r0, nr = _PIECES[j]
            rd = pltpu.make_async_remote_copy(
                src_ref=chunk_buf.at[slot, pl.ds(r0, nr)],
                dst_ref=chunk_buf.at[slot, pl.ds(r0, nr)],
                send_sem=send_sems.at[len(sends)],
                recv_sem=recv_sems.at[j, r ^ (1 << dim)],
                device_id=(nb,),
                device_id_type=pl.DeviceIdType.MESH,
            )
            rd.start()
            sends.append(rd)

        def wait_piece(j, r):
            slot = _lab(my_lab ^ r)
            r0, nr = _PIECES[j]
            rd = pltpu.make_async_remote_copy(
                src_ref=chunk_buf.at[slot, pl.ds(r0, nr)],
                dst_ref=chunk_buf.at[slot, pl.ds(r0, nr)],
                send_sem=send_sems.at[0],
                recv_sem=recv_sems.at[j, r],
                device_id=(my,),
                device_id_type=pl.DeviceIdType.MESH,
            )
            rd.wait_recv()

        for j in range(3):
            r0, nr = _PIECES[j]
            chunk_buf[my, pl.ds(r0, nr)] = x_ref[r0:r0 + nr, :].astype(jnp.bfloat16)
            send_piece(j, 0, j)

        w_bf = w_ref[...].astype(jnp.bfloat16)

        def compute_piece(j, r):
            slot = _lab(my_lab ^ r)
            r0, nr = _PIECES[j]
            y = jnp.dot(chunk_buf[slot, pl.ds(r0, nr)], w_bf,
                        preferred_element_type=jnp.float32)
            y = 0.5 * y * (1.0 + jnp.tanh(_GELU_C * (y + 0.044715 * y * y * y)))
            out_ref[pl.ds(slot * m_per + r0, nr), :] = y

        for j in range(3):
            compute_piece(j, 0)

        for j in range(3):
            j1, j2 = (j + 1) % 3, (j + 2) % 3
            bj = 1 << j
            wait_piece(j, bj)
            send_piece(j, bj, j1)
            send_piece(j, bj, j2)
        for j in range(3):
            compute_piece(j, 1 << j)
        for j in range(3):
            j1, j2 = (j + 1) % 3, (j + 2) % 3
            bj, b1, b2 = 1 << j, 1 << j1, 1 << j2
            wait_piece(j, bj | b1)
            send_piece(j, bj | b1, j2)
            send_piece(j, bj | b1, j)
            wait_piece(j, bj | b2)
            send_piece(j, bj | b2, j)
        for j in range(3):
            j1, j2 = (j + 1) % 3, (j + 2) % 3
            bj = 1 << j
            compute_piece(j, bj | (1 << j1))
            compute_piece(j, bj | (1 << j2))
        for j in range(3):
            wait_piece(j, 7)
            send_piece(j, 7, j)
        for j in range(3):
            j1, j2 = (j + 1) % 3, (j + 2) % 3
            wait_piece(j, 1 << j1)
            wait_piece(j, 1 << j2)
        for j in range(3):
            j1, j2 = (j + 1) % 3, (j + 2) % 3
            compute_piece(j, 7)
            compute_piece(j, 1 << j1)
            compute_piece(j, 1 << j2)
        for j in range(3):
            j1, j2 = (j + 1) % 3, (j + 2) % 3
            wait_piece(j, (1 << j1) | (1 << j2))
            compute_piece(j, (1 << j1) | (1 << j2))
        for rd in sends:
            rd.wait_send()

    out_shape = jax.ShapeDtypeStruct((N_DEV * m_per, n_per), jnp.float32)
    return pl.pallas_call(
        body,
        out_shape=out_shape,
        in_specs=[
            pl.BlockSpec(memory_space=pltpu.VMEM),
            pl.BlockSpec(memory_space=pltpu.VMEM),
        ],
        out_specs=pl.BlockSpec(memory_space=pltpu.VMEM),
        scratch_shapes=[
            pltpu.VMEM((N_DEV, m_per, k), jnp.bfloat16),
            pltpu.SemaphoreType.DMA((21,)),
            pltpu.SemaphoreType.DMA((3, 8)),
        ],
        compiler_params=pltpu.CompilerParams(
            collective_id=0,
            vmem_limit_bytes=100 * 1024 * 1024,
        ),
    )(x, w_mat)


# device time: 126581 ns/iter; 1.3972x vs baseline; 1.0172x over previous
import jax
import jax.numpy as jnp
from jax import lax
from jax.experimental import pallas as pl
from jax.experimental.pallas import tpu as pltpu

N_DEV = 8
_GELU_C = 0.7978845608028654

_PIECES = ((0, 176), (176, 176), (352, 160))
_SUBS = tuple(
    ((r0, 96), (r0 + 96, nr - 96)) for r0, nr in _PIECES
)
NSUB = 2


def _lab(p):
    low = p & 3
    return (p & 4) | (low ^ (low >> 1))


def kernel(x, w_mat):
    m_per, k = x.shape
    _, n_per = w_mat.shape

    def body(x_ref, w_ref, out_ref, chunk_buf, send_sems, recv_sems):
        my = lax.axis_index("i")
        my_lab = _lab(my)

        barrier_sem = pltpu.get_barrier_semaphore()
        for dim in range(3):
            pl.semaphore_signal(
                barrier_sem, inc=1,
                device_id=(_lab(my_lab ^ (1 << dim)),),
                device_id_type=pl.DeviceIdType.MESH,
            )
        pl.semaphore_wait(barrier_sem, 3)

        sends = []

        def send_sub(j, r, dim, u):
            slot = _lab(my_lab ^ r)
            nb = _lab(my_lab ^ (1 << dim))
            r0, nr = _SUBS[j][u]
            rd = pltpu.make_async_remote_copy(
                src_ref=chunk_buf.at[slot, pl.ds(r0, nr)],
                dst_ref=chunk_buf.at[slot, pl.ds(r0, nr)],
                send_sem=send_sems.at[len(sends)],
                recv_sem=recv_sems.at[j, r ^ (1 << dim), u],
                device_id=(nb,),
                device_id_type=pl.DeviceIdType.MESH,
            )
            rd.start()
            sends.append(rd)

        def wait_sub(j, r, u):
            slot = _lab(my_lab ^ r)
            r0, nr = _SUBS[j][u]
            rd = pltpu.make_async_remote_copy(
                src_ref=chunk_buf.at[slot, pl.ds(r0, nr)],
                dst_ref=chunk_buf.at[slot, pl.ds(r0, nr)],
                send_sem=send_sems.at[0],
                recv_sem=recv_sems.at[j, r, u],
                device_id=(my,),
                device_id_type=pl.DeviceIdType.MESH,
            )
            rd.wait_recv()

        for u in range(NSUB):
            for j in range(3):
                r0, nr = _SUBS[j][u]
                chunk_buf[my, pl.ds(r0, nr)] = \
                    x_ref[r0:r0 + nr, :].astype(jnp.bfloat16)
                send_sub(j, 0, j, u)

        w_bf = w_ref[...].astype(jnp.bfloat16)

        def compute_piece(j, r):
            slot = _lab(my_lab ^ r)
            r0, nr = _PIECES[j]
            y = jnp.dot(chunk_buf[slot, pl.ds(r0, nr)], w_bf,
                        preferred_element_type=jnp.float32)
            y = 0.5 * y * (1.0 + jnp.tanh(_GELU_C * (y + 0.044715 * y * y * y)))
            out_ref[pl.ds(slot * m_per + r0, nr), :] = y

        for j in range(3):
            compute_piece(j, 0)

        for u in range(NSUB):
            for j in range(3):
                j1, j2 = (j + 1) % 3, (j + 2) % 3
                bj = 1 << j
                wait_sub(j, bj, u)
                send_sub(j, bj, j1, u)
                send_sub(j, bj, j2, u)
        for j in range(3):
            compute_piece(j, 1 << j)
        for u in range(NSUB):
            for j in range(3):
                j1, j2 = (j + 1) % 3, (j + 2) % 3
                bj, b1, b2 = 1 << j, 1 << j1, 1 << j2
                wait_sub(j, bj | b1, u)
                send_sub(j, bj | b1, j2, u)
                send_sub(j, bj | b1, j, u)
                wait_sub(j, bj | b2, u)
                send_sub(j, bj | b2, j, u)
        for j in range(3):
            j1, j2 = (j + 1) % 3, (j + 2) % 3
            bj = 1 << j
            compute_piece(j, bj | (1 << j1))
            compute_piece(j, bj | (1 << j2))
        for u in range(NSUB):
            for j in range(3):
                wait_sub(j, 7, u)
                send_sub(j, 7, j, u)
        for u in range(NSUB):
            for j in range(3):
                j1, j2 = (j + 1) % 3, (j + 2) % 3
                wait_sub(j, 1 << j1, u)
                wait_sub(j, 1 << j2, u)
        for j in range(3):
            j1, j2 = (j + 1) % 3, (j + 2) % 3
            compute_piece(j, 7)
            compute_piece(j, 1 << j1)
            compute_piece(j, 1 << j2)
        for j in range(3):
            j1, j2 = (j + 1) % 3, (j + 2) % 3
            for u in range(NSUB):
                wait_sub(j, (1 << j1) | (1 << j2), u)
            compute_piece(j, (1 << j1) | (1 << j2))
        for rd in sends:
            rd.wait_send()

    out_shape = jax.ShapeDtypeStruct((N_DEV * m_per, n_per), jnp.float32)
    return pl.pallas_call(
        body,
        out_shape=out_shape,
        in_specs=[
            pl.BlockSpec(memory_space=pltpu.VMEM),
            pl.BlockSpec(memory_space=pltpu.VMEM),
        ],
        out_specs=pl.BlockSpec(memory_space=pltpu.VMEM),
        scratch_shapes=[
            pltpu.VMEM((N_DEV, m_per, k), jnp.bfloat16),
            pltpu.SemaphoreType.DMA((42,)),
            pltpu.SemaphoreType.DMA((3, 8, NSUB)),
        ],
        compiler_params=pltpu.CompilerParams(
            collective_id=0,
            vmem_limit_bytes=100 * 1024 * 1024,
        ),
    )(x, w_mat)
